# Initial kernel scaffold; baseline (speedup 1.0000x reference)
import jax
import jax.numpy as jnp
from jax.experimental import pallas as pl


def kernel(rpn_cls_logits, rpn_bbox_pred, gt_boxes, gt_labels, feat_map_shape):
    raise NotImplementedError("write your pallas kernel here")


# speedup vs baseline: 3.2905x; 3.2905x over previous
import functools

import jax
import jax.numpy as jnp
import numpy as np
from jax.experimental import pallas as pl
from jax.experimental.pallas import tpu as pltpu

_NUM_ANCHORS = 9
_FEAT_STRIDE = 16
_B = 4
_H = 64
_W = 64
_NGT = 50
_A = _H * _W * _NUM_ANCHORS
_ROWS = _A // 128
_RB = 96
_NB = _ROWS // _RB
_TOTAL = float(_B * _A)


def _base_anchors_np(base_size=16, ratios=(0.5, 1.0, 2.0), scales=(8, 16, 32)):
    anchors = []
    cx = base_size / 2.0
    cy = base_size / 2.0
    for r in ratios:
        for s in scales:
            area = float(base_size * s) ** 2
            w = np.sqrt(area / r)
            h = w * r
            anchors.append([cx - 0.5 * w, cy - 0.5 * h, cx + 0.5 * w, cy + 0.5 * h])
    return np.array(anchors, dtype=np.float32)


def _all_anchors_np():
    base = _base_anchors_np()
    shift_x = np.arange(_W, dtype=np.float32) * _FEAT_STRIDE
    shift_y = np.arange(_H, dtype=np.float32) * _FEAT_STRIDE
    sx, sy = np.meshgrid(shift_x, shift_y, indexing="ij")
    shifts = np.reshape(np.stack([sx, sy, sx, sy], axis=-1), [-1, 4])
    all_anchors = base[None, :, :] + shifts[:, None, :]
    return np.reshape(all_anchors, [-1, 4]).astype(np.float32)


def _rpn_body(gt_ref, ax1_ref, ay1_ref, ax2_ref, ay2_ref,
              l0_ref, l1_ref, p0_ref, p1_ref, p2_ref, p3_ref,
              lab_ref, t0_ref, t1_ref, t2_ref, t3_ref, cls_ref, bbox_ref,
              acc_ref):
    b = pl.program_id(0)
    r = pl.program_id(1)

    @pl.when(jnp.logical_and(b == 0, r == 0))
    def _init():
        acc_ref[0] = 0.0
        acc_ref[1] = 0.0
        acc_ref[2] = 0.0

    ax1 = ax1_ref[...]
    ay1 = ay1_ref[...]
    ax2 = ax2_ref[...]
    ay2 = ay2_ref[...]
    area1 = (ax2 - ax1) * (ay2 - ay1)

    neg_inf = jnp.float32(-jnp.inf)
    cur = jnp.full(ax1.shape, neg_inf, jnp.float32)
    mx1 = jnp.zeros(ax1.shape, jnp.float32)
    my1 = jnp.zeros(ax1.shape, jnp.float32)
    mx2 = jnp.zeros(ax1.shape, jnp.float32)
    my2 = jnp.zeros(ax1.shape, jnp.float32)

    for j in range(_NGT):
        gx1 = gt_ref[0, 0, j]
        gy1 = gt_ref[0, 1, j]
        gx2 = gt_ref[0, 2, j]
        gy2 = gt_ref[0, 3, j]
        area2 = gt_ref[0, 4, j]
        valid = gt_ref[0, 5, j]
        x1 = jnp.maximum(ax1, gx1)
        y1 = jnp.maximum(ay1, gy1)
        x2 = jnp.minimum(ax2, gx2)
        y2 = jnp.minimum(ay2, gy2)
        inter = jnp.maximum(x2 - x1, 0.0) * jnp.maximum(y2 - y1, 0.0)
        iou = inter / (area1 + area2 - inter + 1e-08)
        iou = jnp.where(valid > 0.0, iou, -1.0)
        better = iou > cur
        cur = jnp.where(better, iou, cur)
        mx1 = jnp.where(better, gx1, mx1)
        my1 = jnp.where(better, gy1, my1)
        mx2 = jnp.where(better, gx2, mx2)
        my2 = jnp.where(better, gy2, my2)

    pos = cur >= 0.7
    lab_ref[...] = pos.astype(jnp.int32)[None]
    posf = pos.astype(jnp.float32)

    bw = ax2 - ax1 + 1.0
    bh = ay2 - ay1 + 1.0
    bcx = ax1 + 0.5 * bw
    bcy = ay1 + 0.5 * bh
    gw = mx2 - mx1 + 1.0
    gh = my2 - my1 + 1.0
    gcx = mx1 + 0.5 * gw
    gcy = my1 + 0.5 * gh
    t0 = (gcx - bcx) / bw
    t1 = (gcy - bcy) / bh
    t2 = jnp.log(gw / bw)
    t3 = jnp.log(gh / bh)
    t0_ref[...] = t0[None]
    t1_ref[...] = t1[None]
    t2_ref[...] = t2[None]
    t3_ref[...] = t3[None]

    l0 = l0_ref[0]
    l1 = l1_ref[0]
    m = jnp.maximum(l0, l1)
    lse = m + jnp.log(jnp.exp(l0 - m) + jnp.exp(l1 - m))
    ce = lse - jnp.where(pos, l1, l0)
    acc_ref[0] = acc_ref[0] + jnp.sum(ce)

    sl1 = jnp.zeros(ax1.shape, jnp.float32)
    for p_ref, t in ((p0_ref, t0), (p1_ref, t1), (p2_ref, t2), (p3_ref, t3)):
        d = p_ref[0] - t
        ad = jnp.abs(d)
        sl1 = sl1 + jnp.where(ad < 1.0, 0.5 * d * d, ad - 0.5)
    acc_ref[1] = acc_ref[1] + jnp.sum(sl1 * posf)
    acc_ref[2] = acc_ref[2] + jnp.sum(posf)

    cls_ref[0, 0] = acc_ref[0] / _TOTAL
    bbox_ref[0, 0] = acc_ref[1] / jnp.maximum(acc_ref[2], 1.0)


@jax.jit
def _run(rpn_cls_logits, rpn_bbox_pred, gt_boxes, gt_labels):
    anchors = _all_anchors_np()
    ax1 = jnp.asarray(anchors[:, 0].reshape(_ROWS, 128))
    ay1 = jnp.asarray(anchors[:, 1].reshape(_ROWS, 128))
    ax2 = jnp.asarray(anchors[:, 2].reshape(_ROWS, 128))
    ay2 = jnp.asarray(anchors[:, 3].reshape(_ROWS, 128))

    l0 = rpn_cls_logits[:, :, 0].reshape(_B, _ROWS, 128)
    l1 = rpn_cls_logits[:, :, 1].reshape(_B, _ROWS, 128)
    pred = rpn_bbox_pred.reshape(_B, _A, 4)
    p0 = pred[:, :, 0].reshape(_B, _ROWS, 128)
    p1 = pred[:, :, 1].reshape(_B, _ROWS, 128)
    p2 = pred[:, :, 2].reshape(_B, _ROWS, 128)
    p3 = pred[:, :, 3].reshape(_B, _ROWS, 128)

    area2 = (gt_boxes[:, :, 2] - gt_boxes[:, :, 0]) * (gt_boxes[:, :, 3] - gt_boxes[:, :, 1])
    valid = (gt_labels > 0).astype(jnp.float32)
    comps = jnp.stack(
        [gt_boxes[:, :, 0], gt_boxes[:, :, 1], gt_boxes[:, :, 2], gt_boxes[:, :, 3],
         area2, valid], axis=1)
    garr = jnp.pad(comps, ((0, 0), (0, 2), (0, 14)))

    anchor_spec = pl.BlockSpec((_RB, 128), lambda b, r: (r, 0))
    batch_spec = pl.BlockSpec((1, _RB, 128), lambda b, r: (b, r, 0))
    scalar_out = pl.BlockSpec((1, 1), lambda b, r: (0, 0), memory_space=pltpu.SMEM)

    out = pl.pallas_call(
        _rpn_body,
        grid=(_B, _NB),
        in_specs=[
            pl.BlockSpec((1, 8, 64), lambda b, r: (b, 0, 0), memory_space=pltpu.SMEM),
            anchor_spec, anchor_spec, anchor_spec, anchor_spec,
            batch_spec, batch_spec,
            batch_spec, batch_spec, batch_spec, batch_spec,
        ],
        out_specs=[
            batch_spec, batch_spec, batch_spec, batch_spec, batch_spec,
            scalar_out, scalar_out,
        ],
        out_shape=[
            jax.ShapeDtypeStruct((_B, _ROWS, 128), jnp.int32),
            jax.ShapeDtypeStruct((_B, _ROWS, 128), jnp.float32),
            jax.ShapeDtypeStruct((_B, _ROWS, 128), jnp.float32),
            jax.ShapeDtypeStruct((_B, _ROWS, 128), jnp.float32),
            jax.ShapeDtypeStruct((_B, _ROWS, 128), jnp.float32),
            jax.ShapeDtypeStruct((1, 1), jnp.float32),
            jax.ShapeDtypeStruct((1, 1), jnp.float32),
        ],
        scratch_shapes=[pltpu.SMEM((4,), jnp.float32)],
        compiler_params=pltpu.CompilerParams(
            dimension_semantics=("arbitrary", "arbitrary")),
    )(garr, ax1, ay1, ax2, ay2, l0, l1, p0, p1, p2, p3)

    lab, t0, t1, t2, t3, cls_l, bbox_l = out
    rpn_labels = lab.reshape(_B, _A)
    rpn_targets = jnp.stack(
        [t0.reshape(_B, _A), t1.reshape(_B, _A), t2.reshape(_B, _A), t3.reshape(_B, _A)],
        axis=-1)
    return cls_l[0, 0], bbox_l[0, 0], rpn_labels, rpn_targets


def kernel(rpn_cls_logits, rpn_bbox_pred, gt_boxes, gt_labels, feat_map_shape):
    return _run(rpn_cls_logits, rpn_bbox_pred, gt_boxes, gt_labels)


# speedup vs baseline: 3.4714x; 1.0550x over previous
import jax
import jax.numpy as jnp
import numpy as np
from jax import lax
from jax.experimental import pallas as pl
from jax.experimental.pallas import tpu as pltpu

_NUM_ANCHORS = 9
_FEAT_STRIDE = 16
_B = 4
_H = 64
_W = 64
_NGT = 50
_A = _H * _W * _NUM_ANCHORS
_ROWS = _A // 128
_RB = 96
_NB = _ROWS // _RB
_TOTAL = float(_B * _A)


def _base_anchors_np(base_size=16, ratios=(0.5, 1.0, 2.0), scales=(8, 16, 32)):
    anchors = []
    cx = base_size / 2.0
    cy = base_size / 2.0
    for r in ratios:
        for s in scales:
            area = float(base_size * s) ** 2
            w = np.sqrt(area / r)
            h = w * r
            anchors.append([cx - 0.5 * w, cy - 0.5 * h, cx + 0.5 * w, cy + 0.5 * h])
    return np.array(anchors, dtype=np.float32)


def _all_anchors_np():
    base = _base_anchors_np()
    shift_x = np.arange(_W, dtype=np.float32) * _FEAT_STRIDE
    shift_y = np.arange(_H, dtype=np.float32) * _FEAT_STRIDE
    sx, sy = np.meshgrid(shift_x, shift_y, indexing="ij")
    shifts = np.reshape(np.stack([sx, sy, sx, sy], axis=-1), [-1, 4])
    all_anchors = base[None, :, :] + shifts[:, None, :]
    return np.reshape(all_anchors, [-1, 4]).astype(np.float32)


def _interleave_mats_np():
    e4 = np.zeros((512, 512), np.float32)
    for c in range(4):
        for la in range(128):
            e4[c * 128 + la, 4 * la + c] = 1.0
    s2 = np.zeros((256, 256), np.float32)
    for j in range(256):
        s2[j ^ 1, j] = 1.0
    p2 = np.zeros((128, 256), np.float32)
    p4 = np.zeros((128, 512), np.float32)
    for la in range(128):
        p2[la, 2 * la] = 1.0
        p2[la, 2 * la + 1] = 1.0
        for c in range(4):
            p4[la, 4 * la + c] = 1.0
    return e4, s2, p2, p4


def _rpn_body(gt_ref, ax1_ref, ay1_ref, ax2_ref, ay2_ref,
              lg_ref, pr_ref, e4_ref, s2_ref, p2_ref, p4_ref,
              lab_ref, tg_ref, cls_ref, bbox_ref,
              acc_ref):
    b = pl.program_id(0)
    r = pl.program_id(1)

    @pl.when(jnp.logical_and(b == 0, r == 0))
    def _init():
        acc_ref[0] = 0.0
        acc_ref[1] = 0.0
        acc_ref[2] = 0.0

    ax1 = ax1_ref[...]
    ay1 = ay1_ref[...]
    ax2 = ax2_ref[...]
    ay2 = ay2_ref[...]
    area1 = (ax2 - ax1) * (ay2 - ay1)

    neg_inf = jnp.float32(-jnp.inf)
    cur = jnp.full(ax1.shape, neg_inf, jnp.float32)
    mx1 = jnp.zeros(ax1.shape, jnp.float32)
    my1 = jnp.zeros(ax1.shape, jnp.float32)
    mx2 = jnp.zeros(ax1.shape, jnp.float32)
    my2 = jnp.zeros(ax1.shape, jnp.float32)

    for j in range(_NGT):
        gx1 = gt_ref[0, 0, j]
        gy1 = gt_ref[0, 1, j]
        gx2 = gt_ref[0, 2, j]
        gy2 = gt_ref[0, 3, j]
        area2 = gt_ref[0, 4, j]
        valid = gt_ref[0, 5, j]
        x1 = jnp.maximum(ax1, gx1)
        y1 = jnp.maximum(ay1, gy1)
        x2 = jnp.minimum(ax2, gx2)
        y2 = jnp.minimum(ay2, gy2)
        inter = jnp.maximum(x2 - x1, 0.0) * jnp.maximum(y2 - y1, 0.0)
        iou = inter / (area1 + area2 - inter + 1e-08)
        iou = jnp.where(valid > 0.0, iou, -1.0)
        better = iou > cur
        cur = jnp.where(better, iou, cur)
        mx1 = jnp.where(better, gx1, mx1)
        my1 = jnp.where(better, gy1, my1)
        mx2 = jnp.where(better, gx2, mx2)
        my2 = jnp.where(better, gy2, my2)

    pos = cur >= 0.7
    lab_ref[...] = pos.astype(jnp.int32)[None]
    posf = pos.astype(jnp.float32)

    bw = ax2 - ax1 + 1.0
    bh = ay2 - ay1 + 1.0
    bcx = ax1 + 0.5 * bw
    bcy = ay1 + 0.5 * bh
    gw = mx2 - mx1 + 1.0
    gh = my2 - my1 + 1.0
    gcx = mx1 + 0.5 * gw
    gcy = my1 + 0.5 * gh
    t0 = (gcx - bcx) / bw
    t1 = (gcy - bcy) / bh
    t2 = jnp.log(gw / bw)
    t3 = jnp.log(gh / bh)

    g = jnp.concatenate([t0, t1, t2, t3], axis=1)
    v = jnp.dot(g, e4_ref[...], preferred_element_type=jnp.float32)
    tg_ref[...] = v[None]

    lg = lg_ref[0]
    evencol = (lax.broadcasted_iota(jnp.int32, (_RB, 256), 1) % 2) == 0
    lsw = jnp.dot(lg, s2_ref[...], preferred_element_type=jnp.float32)
    m = jnp.maximum(lg, lsw)
    lse = m + jnp.log(jnp.exp(lg - m) + jnp.exp(lsw - m))
    pos_x2 = jnp.dot(posf, p2_ref[...], preferred_element_type=jnp.float32)
    ce = lse - lg - pos_x2 * (lsw - lg)
    acc_ref[0] = acc_ref[0] + jnp.sum(jnp.where(evencol, ce, 0.0))

    d = pr_ref[0] - v
    ad = jnp.abs(d)
    f = jnp.where(ad < 1.0, 0.5 * d * d, ad - 0.5)
    pos_x4 = jnp.dot(posf, p4_ref[...], preferred_element_type=jnp.float32)
    acc_ref[1] = acc_ref[1] + jnp.sum(f * pos_x4)
    acc_ref[2] = acc_ref[2] + jnp.sum(posf)

    cls_ref[0, 0] = acc_ref[0] / _TOTAL
    bbox_ref[0, 0] = acc_ref[1] / jnp.maximum(acc_ref[2], 1.0)


@jax.jit
def _run(rpn_cls_logits, rpn_bbox_pred, gt_boxes, gt_labels):
    anchors = _all_anchors_np()
    ax1 = jnp.asarray(anchors[:, 0].reshape(_ROWS, 128))
    ay1 = jnp.asarray(anchors[:, 1].reshape(_ROWS, 128))
    ax2 = jnp.asarray(anchors[:, 2].reshape(_ROWS, 128))
    ay2 = jnp.asarray(anchors[:, 3].reshape(_ROWS, 128))

    lg = rpn_cls_logits.reshape(_B, _ROWS, 256)
    pr = rpn_bbox_pred.reshape(_B, _ROWS, 512)

    area2 = (gt_boxes[:, :, 2] - gt_boxes[:, :, 0]) * (gt_boxes[:, :, 3] - gt_boxes[:, :, 1])
    valid = (gt_labels > 0).astype(jnp.float32)
    comps = jnp.stack(
        [gt_boxes[:, :, 0], gt_boxes[:, :, 1], gt_boxes[:, :, 2], gt_boxes[:, :, 3],
         area2, valid], axis=1)
    garr = jnp.pad(comps, ((0, 0), (0, 2), (0, 14)))

    e4_np, s2_np, p2_np, p4_np = _interleave_mats_np()
    e4 = jnp.asarray(e4_np)
    s2 = jnp.asarray(s2_np)
    p2 = jnp.asarray(p2_np)
    p4 = jnp.asarray(p4_np)

    anchor_spec = pl.BlockSpec((_RB, 128), lambda b, r: (r, 0))
    lab_spec = pl.BlockSpec((1, _RB, 128), lambda b, r: (b, r, 0))
    lg_spec = pl.BlockSpec((1, _RB, 256), lambda b, r: (b, r, 0))
    pr_spec = pl.BlockSpec((1, _RB, 512), lambda b, r: (b, r, 0))
    scalar_out = pl.BlockSpec((1, 1), lambda b, r: (0, 0), memory_space=pltpu.SMEM)

    out = pl.pallas_call(
        _rpn_body,
        grid=(_B, _NB),
        in_specs=[
            pl.BlockSpec((1, 8, 64), lambda b, r: (b, 0, 0), memory_space=pltpu.SMEM),
            anchor_spec, anchor_spec, anchor_spec, anchor_spec,
            lg_spec, pr_spec,
            pl.BlockSpec((512, 512), lambda b, r: (0, 0)),
            pl.BlockSpec((256, 256), lambda b, r: (0, 0)),
            pl.BlockSpec((128, 256), lambda b, r: (0, 0)),
            pl.BlockSpec((128, 512), lambda b, r: (0, 0)),
        ],
        out_specs=[
            lab_spec, pr_spec,
            scalar_out, scalar_out,
        ],
        out_shape=[
            jax.ShapeDtypeStruct((_B, _ROWS, 128), jnp.int32),
            jax.ShapeDtypeStruct((_B, _ROWS, 512), jnp.float32),
            jax.ShapeDtypeStruct((1, 1), jnp.float32),
            jax.ShapeDtypeStruct((1, 1), jnp.float32),
        ],
        scratch_shapes=[pltpu.SMEM((4,), jnp.float32)],
        compiler_params=pltpu.CompilerParams(
            dimension_semantics=("arbitrary", "arbitrary")),
    )(garr, ax1, ay1, ax2, ay2, lg, pr, e4, s2, p2, p4)

    lab, tg, cls_l, bbox_l = out
    rpn_labels = lab.reshape(_B, _A)
    rpn_targets = tg.reshape(_B, _A, 4)
    return cls_l[0, 0], bbox_l[0, 0], rpn_labels, rpn_targets


def kernel(rpn_cls_logits, rpn_bbox_pred, gt_boxes, gt_labels, feat_map_shape):
    return _run(rpn_cls_logits, rpn_bbox_pred, gt_boxes, gt_labels)


# speedup vs baseline: 5.6748x; 1.6348x over previous
import jax
import jax.numpy as jnp
import numpy as np
from jax import lax
from jax.experimental import pallas as pl
from jax.experimental.pallas import tpu as pltpu

_NUM_ANCHORS = 9
_FEAT_STRIDE = 16
_B = 4
_H = 64
_W = 64
_NGT = 50
_A = _H * _W * _NUM_ANCHORS
_ROWS = _A // 128
_RB = 96
_NB = _ROWS // _RB
_TOTAL = float(_B * _A)


def _base_anchors_np(base_size=16, ratios=(0.5, 1.0, 2.0), scales=(8, 16, 32)):
    anchors = []
    cx = base_size / 2.0
    cy = base_size / 2.0
    for r in ratios:
        for s in scales:
            area = float(base_size * s) ** 2
            w = np.sqrt(area / r)
            h = w * r
            anchors.append([cx - 0.5 * w, cy - 0.5 * h, cx + 0.5 * w, cy + 0.5 * h])
    return np.array(anchors, dtype=np.float32)


def _all_anchors_np():
    base = _base_anchors_np()
    shift_x = np.arange(_W, dtype=np.float32) * _FEAT_STRIDE
    shift_y = np.arange(_H, dtype=np.float32) * _FEAT_STRIDE
    sx, sy = np.meshgrid(shift_x, shift_y, indexing="ij")
    shifts = np.reshape(np.stack([sx, sy, sx, sy], axis=-1), [-1, 4])
    all_anchors = base[None, :, :] + shifts[:, None, :]
    return np.reshape(all_anchors, [-1, 4]).astype(np.float32)


def _rpn_body(gt_ref, ax1_ref, ay1_ref, ax2_ref, ay2_ref,
              lg_ref, pr_ref,
              lab_ref, tg_ref, cls_ref, bbox_ref,
              acc_ref):
    b = pl.program_id(0)
    r = pl.program_id(1)

    @pl.when(jnp.logical_and(b == 0, r == 0))
    def _init():
        acc_ref[0] = 0.0
        acc_ref[1] = 0.0
        acc_ref[2] = 0.0

    ax1 = ax1_ref[...]
    ay1 = ay1_ref[...]
    ax2 = ax2_ref[...]
    ay2 = ay2_ref[...]
    area1 = (ax2 - ax1) * (ay2 - ay1)

    neg_inf = jnp.float32(-jnp.inf)
    cur = jnp.full(ax1.shape, neg_inf, jnp.float32)
    mx1 = jnp.zeros(ax1.shape, jnp.float32)
    my1 = jnp.zeros(ax1.shape, jnp.float32)
    mx2 = jnp.zeros(ax1.shape, jnp.float32)
    my2 = jnp.zeros(ax1.shape, jnp.float32)

    for j in range(_NGT):
        gx1 = gt_ref[0, 0, j]
        gy1 = gt_ref[0, 1, j]
        gx2 = gt_ref[0, 2, j]
        gy2 = gt_ref[0, 3, j]
        area2 = gt_ref[0, 4, j]
        valid = gt_ref[0, 5, j]
        x1 = jnp.maximum(ax1, gx1)
        y1 = jnp.maximum(ay1, gy1)
        x2 = jnp.minimum(ax2, gx2)
        y2 = jnp.minimum(ay2, gy2)
        inter = jnp.maximum(x2 - x1, 0.0) * jnp.maximum(y2 - y1, 0.0)
        iou = inter / (area1 + area2 - inter + 1e-08)
        iou = jnp.where(valid > 0.0, iou, -1.0)
        better = iou > cur
        cur = jnp.where(better, iou, cur)
        mx1 = jnp.where(better, gx1, mx1)
        my1 = jnp.where(better, gy1, my1)
        mx2 = jnp.where(better, gx2, mx2)
        my2 = jnp.where(better, gy2, my2)

    pos = cur >= 0.7
    lab_ref[...] = pos.astype(jnp.int32)[None]
    posf = pos.astype(jnp.float32)

    bw = ax2 - ax1 + 1.0
    bh = ay2 - ay1 + 1.0
    bcx = ax1 + 0.5 * bw
    bcy = ay1 + 0.5 * bh
    gw = mx2 - mx1 + 1.0
    gh = my2 - my1 + 1.0
    gcx = mx1 + 0.5 * gw
    gcy = my1 + 0.5 * gh
    t0 = (gcx - bcx) / bw
    t1 = (gcy - bcy) / bh
    t2 = jnp.log(gw / bw)
    t3 = jnp.log(gh / bh)

    v = jnp.stack([t0, t1, t2, t3], axis=1)
    tg_ref[...] = jnp.reshape(v, (4 * _RB, 128))[None]

    l3 = jnp.reshape(lg_ref[0], (_RB, 2, 128))
    l0 = l3[:, 0, :]
    l1 = l3[:, 1, :]
    m = jnp.maximum(l0, l1)
    lse = m + jnp.log(jnp.exp(l0 - m) + jnp.exp(l1 - m))
    ce = lse - jnp.where(pos, l1, l0)
    acc_ref[0] = acc_ref[0] + jnp.sum(ce)

    p4 = jnp.reshape(pr_ref[0], (_RB, 4, 128))
    d = p4 - v
    ad = jnp.abs(d)
    f = jnp.where(ad < 1.0, 0.5 * d * d, ad - 0.5)
    sl1 = f[:, 0, :] + f[:, 1, :] + f[:, 2, :] + f[:, 3, :]
    acc_ref[1] = acc_ref[1] + jnp.sum(sl1 * posf)
    acc_ref[2] = acc_ref[2] + jnp.sum(posf)

    cls_ref[0, 0] = acc_ref[0] / _TOTAL
    bbox_ref[0, 0] = acc_ref[1] / jnp.maximum(acc_ref[2], 1.0)


@jax.jit
def _run(rpn_cls_logits, rpn_bbox_pred, gt_boxes, gt_labels):
    anchors = _all_anchors_np()
    ax1 = jnp.asarray(anchors[:, 0].reshape(_ROWS, 128))
    ay1 = jnp.asarray(anchors[:, 1].reshape(_ROWS, 128))
    ax2 = jnp.asarray(anchors[:, 2].reshape(_ROWS, 128))
    ay2 = jnp.asarray(anchors[:, 3].reshape(_ROWS, 128))

    lg = (rpn_cls_logits.reshape(_B, _ROWS, 128, 2)
          .transpose(0, 1, 3, 2).reshape(_B, 2 * _ROWS, 128))
    pr = (rpn_bbox_pred.reshape(_B, _A, 4).reshape(_B, _ROWS, 128, 4)
          .transpose(0, 1, 3, 2).reshape(_B, 4 * _ROWS, 128))

    area2 = (gt_boxes[:, :, 2] - gt_boxes[:, :, 0]) * (gt_boxes[:, :, 3] - gt_boxes[:, :, 1])
    valid = (gt_labels > 0).astype(jnp.float32)
    comps = jnp.stack(
        [gt_boxes[:, :, 0], gt_boxes[:, :, 1], gt_boxes[:, :, 2], gt_boxes[:, :, 3],
         area2, valid], axis=1)
    garr = jnp.pad(comps, ((0, 0), (0, 2), (0, 14)))

    anchor_spec = pl.BlockSpec((_RB, 128), lambda b, r: (r, 0))
    lab_spec = pl.BlockSpec((1, _RB, 128), lambda b, r: (b, r, 0))
    lg_spec = pl.BlockSpec((1, 2 * _RB, 128), lambda b, r: (b, r, 0))
    pr_spec = pl.BlockSpec((1, 4 * _RB, 128), lambda b, r: (b, r, 0))
    scalar_out = pl.BlockSpec((1, 1), lambda b, r: (0, 0), memory_space=pltpu.SMEM)

    out = pl.pallas_call(
        _rpn_body,
        grid=(_B, _NB),
        in_specs=[
            pl.BlockSpec((1, 8, 64), lambda b, r: (b, 0, 0), memory_space=pltpu.SMEM),
            anchor_spec, anchor_spec, anchor_spec, anchor_spec,
            lg_spec, pr_spec,
        ],
        out_specs=[
            lab_spec, pr_spec,
            scalar_out, scalar_out,
        ],
        out_shape=[
            jax.ShapeDtypeStruct((_B, _ROWS, 128), jnp.int32),
            jax.ShapeDtypeStruct((_B, 4 * _ROWS, 128), jnp.float32),
            jax.ShapeDtypeStruct((1, 1), jnp.float32),
            jax.ShapeDtypeStruct((1, 1), jnp.float32),
        ],
        scratch_shapes=[pltpu.SMEM((4,), jnp.float32)],
        compiler_params=pltpu.CompilerParams(
            dimension_semantics=("arbitrary", "arbitrary")),
    )(garr, ax1, ay1, ax2, ay2, lg, pr)

    lab, tg, cls_l, bbox_l = out
    rpn_labels = lab.reshape(_B, _A)
    rpn_targets = (tg.reshape(_B, _ROWS, 4, 128)
                   .transpose(0, 1, 3, 2).reshape(_B, _A, 4))
    return cls_l[0, 0], bbox_l[0, 0], rpn_labels, rpn_targets


def kernel(rpn_cls_logits, rpn_bbox_pred, gt_boxes, gt_labels, feat_map_shape):
    return _run(rpn_cls_logits, rpn_bbox_pred, gt_boxes, gt_labels)


# speedup vs baseline: 11.8155x; 2.0821x over previous
import jax
import jax.numpy as jnp
import numpy as np
from jax import lax
from jax.experimental import pallas as pl
from jax.experimental.pallas import tpu as pltpu

_NUM_ANCHORS = 9
_FEAT_STRIDE = 16
_B = 4
_H = 64
_W = 64
_NGT = 50
_A = _H * _W * _NUM_ANCHORS
_ROWS = _A // 128
_RB = 96
_NB = _ROWS // _RB
_TOTAL = float(_B * _A)


def _base_anchors_np(base_size=16, ratios=(0.5, 1.0, 2.0), scales=(8, 16, 32)):
    anchors = []
    cx = base_size / 2.0
    cy = base_size / 2.0
    for r in ratios:
        for s in scales:
            area = float(base_size * s) ** 2
            w = np.sqrt(area / r)
            h = w * r
            anchors.append([cx - 0.5 * w, cy - 0.5 * h, cx + 0.5 * w, cy + 0.5 * h])
    return np.array(anchors, dtype=np.float32)


def _all_anchors_np():
    base = _base_anchors_np()
    shift_x = np.arange(_W, dtype=np.float32) * _FEAT_STRIDE
    shift_y = np.arange(_H, dtype=np.float32) * _FEAT_STRIDE
    sx, sy = np.meshgrid(shift_x, shift_y, indexing="ij")
    shifts = np.reshape(np.stack([sx, sy, sx, sy], axis=-1), [-1, 4])
    all_anchors = base[None, :, :] + shifts[:, None, :]
    return np.reshape(all_anchors, [-1, 4]).astype(np.float32)


def _interleave_mats_np():
    e4 = np.zeros((512, 512), np.float32)
    for c in range(4):
        for la in range(128):
            e4[c * 128 + la, 4 * la + c] = 1.0
    p4 = np.zeros((128, 512), np.float32)
    for la in range(128):
        for c in range(4):
            p4[la, 4 * la + c] = 1.0
    return e4, p4


def _rpn_body(gt_ref, ax1_ref, ay1_ref, ax2_ref, ay2_ref,
              lg_ref, pr_ref, e4_ref, p4_ref,
              lab_ref, tg_ref, cls_ref, bbox_ref,
              acc_ref):
    b = pl.program_id(0)
    r = pl.program_id(1)

    @pl.when(jnp.logical_and(b == 0, r == 0))
    def _init():
        acc_ref[0] = 0.0
        acc_ref[1] = 0.0
        acc_ref[2] = 0.0

    ax1 = ax1_ref[...]
    ay1 = ay1_ref[...]
    ax2 = ax2_ref[...]
    ay2 = ay2_ref[...]
    area1 = (ax2 - ax1) * (ay2 - ay1)

    neg_inf = jnp.float32(-jnp.inf)
    cur = jnp.full(ax1.shape, neg_inf, jnp.float32)
    mx1 = jnp.zeros(ax1.shape, jnp.float32)
    my1 = jnp.zeros(ax1.shape, jnp.float32)
    mx2 = jnp.zeros(ax1.shape, jnp.float32)
    my2 = jnp.zeros(ax1.shape, jnp.float32)

    for j in range(_NGT):
        gx1 = gt_ref[0, 0, j]
        gy1 = gt_ref[0, 1, j]
        gx2 = gt_ref[0, 2, j]
        gy2 = gt_ref[0, 3, j]
        area2 = gt_ref[0, 4, j]
        valid = gt_ref[0, 5, j]
        x1 = jnp.maximum(ax1, gx1)
        y1 = jnp.maximum(ay1, gy1)
        x2 = jnp.minimum(ax2, gx2)
        y2 = jnp.minimum(ay2, gy2)
        inter = jnp.maximum(x2 - x1, 0.0) * jnp.maximum(y2 - y1, 0.0)
        iou = inter / (area1 + area2 - inter + 1e-08)
        iou = jnp.where(valid > 0.0, iou, -1.0)
        better = iou > cur
        cur = jnp.where(better, iou, cur)
        mx1 = jnp.where(better, gx1, mx1)
        my1 = jnp.where(better, gy1, my1)
        mx2 = jnp.where(better, gx2, mx2)
        my2 = jnp.where(better, gy2, my2)

    pos = cur >= 0.7
    lab_ref[...] = pos.astype(jnp.int32)[None]
    posf = pos.astype(jnp.float32)

    bw = ax2 - ax1 + 1.0
    bh = ay2 - ay1 + 1.0
    bcx = ax1 + 0.5 * bw
    bcy = ay1 + 0.5 * bh
    gw = mx2 - mx1 + 1.0
    gh = my2 - my1 + 1.0
    gcx = mx1 + 0.5 * gw
    gcy = my1 + 0.5 * gh
    t0 = (gcx - bcx) / bw
    t1 = (gcy - bcy) / bh
    t2 = jnp.log(gw / bw)
    t3 = jnp.log(gh / bh)

    v = jnp.stack([t0, t1, t2, t3], axis=1)
    tg_ref[...] = jnp.reshape(v, (4 * _RB, 128))[None]

    l3 = jnp.reshape(lg_ref[0], (_RB, 2, 128))
    l0 = l3[:, 0, :]
    l1 = l3[:, 1, :]
    m = jnp.maximum(l0, l1)
    lse = m + jnp.log(jnp.exp(l0 - m) + jnp.exp(l1 - m))
    ce = lse - jnp.where(pos, l1, l0)
    acc_ref[0] = acc_ref[0] + jnp.sum(ce)

    g = jnp.concatenate([t0, t1, t2, t3], axis=1)
    vl = jnp.dot(g, e4_ref[...], preferred_element_type=jnp.float32,
                 precision=lax.Precision.HIGHEST)
    d = pr_ref[0] - vl
    ad = jnp.abs(d)
    f = jnp.where(ad < 1.0, 0.5 * d * d, ad - 0.5)
    pos_x4 = jnp.dot(posf, p4_ref[...], preferred_element_type=jnp.float32)
    acc_ref[1] = acc_ref[1] + jnp.sum(f * pos_x4)
    acc_ref[2] = acc_ref[2] + jnp.sum(posf)

    cls_ref[0, 0] = acc_ref[0] / _TOTAL
    bbox_ref[0, 0] = acc_ref[1] / jnp.maximum(acc_ref[2], 1.0)


@jax.jit
def _run(rpn_cls_logits, rpn_bbox_pred, gt_boxes, gt_labels):
    anchors = _all_anchors_np()
    ax1 = jnp.asarray(anchors[:, 0].reshape(_ROWS, 128))
    ay1 = jnp.asarray(anchors[:, 1].reshape(_ROWS, 128))
    ax2 = jnp.asarray(anchors[:, 2].reshape(_ROWS, 128))
    ay2 = jnp.asarray(anchors[:, 3].reshape(_ROWS, 128))

    lg = (rpn_cls_logits.reshape(_B, _ROWS, 128, 2)
          .transpose(0, 1, 3, 2).reshape(_B, 2 * _ROWS, 128))
    pr = rpn_bbox_pred.reshape(_B, _ROWS, 512)

    area2 = (gt_boxes[:, :, 2] - gt_boxes[:, :, 0]) * (gt_boxes[:, :, 3] - gt_boxes[:, :, 1])
    valid = (gt_labels > 0).astype(jnp.float32)
    comps = jnp.stack(
        [gt_boxes[:, :, 0], gt_boxes[:, :, 1], gt_boxes[:, :, 2], gt_boxes[:, :, 3],
         area2, valid], axis=1)
    garr = jnp.pad(comps, ((0, 0), (0, 2), (0, 14)))

    e4_np, p4_np = _interleave_mats_np()
    e4 = jnp.asarray(e4_np)
    p4 = jnp.asarray(p4_np)

    anchor_spec = pl.BlockSpec((_RB, 128), lambda b, r: (r, 0))
    lab_spec = pl.BlockSpec((1, _RB, 128), lambda b, r: (b, r, 0))
    lg_spec = pl.BlockSpec((1, 2 * _RB, 128), lambda b, r: (b, r, 0))
    pr_spec = pl.BlockSpec((1, _RB, 512), lambda b, r: (b, r, 0))
    tg_spec = pl.BlockSpec((1, 4 * _RB, 128), lambda b, r: (b, r, 0))
    scalar_out = pl.BlockSpec((1, 1), lambda b, r: (0, 0), memory_space=pltpu.SMEM)

    out = pl.pallas_call(
        _rpn_body,
        grid=(_B, _NB),
        in_specs=[
            pl.BlockSpec((1, 8, 64), lambda b, r: (b, 0, 0), memory_space=pltpu.SMEM),
            anchor_spec, anchor_spec, anchor_spec, anchor_spec,
            lg_spec, pr_spec,
            pl.BlockSpec((512, 512), lambda b, r: (0, 0)),
            pl.BlockSpec((128, 512), lambda b, r: (0, 0)),
        ],
        out_specs=[
            lab_spec, tg_spec,
            scalar_out, scalar_out,
        ],
        out_shape=[
            jax.ShapeDtypeStruct((_B, _ROWS, 128), jnp.int32),
            jax.ShapeDtypeStruct((_B, 4 * _ROWS, 128), jnp.float32),
            jax.ShapeDtypeStruct((1, 1), jnp.float32),
            jax.ShapeDtypeStruct((1, 1), jnp.float32),
        ],
        scratch_shapes=[pltpu.SMEM((4,), jnp.float32)],
        compiler_params=pltpu.CompilerParams(
            dimension_semantics=("arbitrary", "arbitrary")),
    )(garr, ax1, ay1, ax2, ay2, lg, pr, e4, p4)

    lab, tg, cls_l, bbox_l = out
    rpn_labels = lab.reshape(_B, _A)
    rpn_targets = (tg.reshape(_B, _ROWS, 4, 128)
                   .transpose(0, 1, 3, 2).reshape(_B, _A, 4))
    return cls_l[0, 0], bbox_l[0, 0], rpn_labels, rpn_targets


def kernel(rpn_cls_logits, rpn_bbox_pred, gt_boxes, gt_labels, feat_map_shape):
    return _run(rpn_cls_logits, rpn_bbox_pred, gt_boxes, gt_labels)


# speedup vs baseline: 11.8446x; 1.0025x over previous
import jax
import jax.numpy as jnp
import numpy as np
from jax import lax
from jax.experimental import pallas as pl
from jax.experimental.pallas import tpu as pltpu

_NUM_ANCHORS = 9
_FEAT_STRIDE = 16
_B = 4
_H = 64
_W = 64
_NGT = 50
_A = _H * _W * _NUM_ANCHORS
_ROWS = _A // 128
_RB = 96
_NB = _ROWS // _RB
_TOTAL = float(_B * _A)


def _base_anchors_np(base_size=16, ratios=(0.5, 1.0, 2.0), scales=(8, 16, 32)):
    anchors = []
    cx = base_size / 2.0
    cy = base_size / 2.0
    for r in ratios:
        for s in scales:
            area = float(base_size * s) ** 2
            w = np.sqrt(area / r)
            h = w * r
            anchors.append([cx - 0.5 * w, cy - 0.5 * h, cx + 0.5 * w, cy + 0.5 * h])
    return np.array(anchors, dtype=np.float32)


def _all_anchors_np():
    base = _base_anchors_np()
    shift_x = np.arange(_W, dtype=np.float32) * _FEAT_STRIDE
    shift_y = np.arange(_H, dtype=np.float32) * _FEAT_STRIDE
    sx, sy = np.meshgrid(shift_x, shift_y, indexing="ij")
    shifts = np.reshape(np.stack([sx, sy, sx, sy], axis=-1), [-1, 4])
    all_anchors = base[None, :, :] + shifts[:, None, :]
    return np.reshape(all_anchors, [-1, 4]).astype(np.float32)


def _interleave_mats_np():
    e4 = np.zeros((512, 512), np.float32)
    for c in range(4):
        for la in range(128):
            e4[c * 128 + la, 4 * la + c] = 1.0
    p4 = np.zeros((128, 512), np.float32)
    for la in range(128):
        for c in range(4):
            p4[la, 4 * la + c] = 1.0
    return e4, p4


def _rpn_body(gt_ref, ax1_ref, ay1_ref, ax2_ref, ay2_ref,
              lg_ref, pr_ref, e4_ref, p4_ref,
              lab_ref, tg_ref, cls_ref, bbox_ref,
              acc_ref):
    b = pl.program_id(0)
    r = pl.program_id(1)

    @pl.when(jnp.logical_and(b == 0, r == 0))
    def _init():
        acc_ref[0] = 0.0
        acc_ref[1] = 0.0
        acc_ref[2] = 0.0

    ax1 = ax1_ref[...]
    ay1 = ay1_ref[...]
    ax2 = ax2_ref[...]
    ay2 = ay2_ref[...]
    area1 = (ax2 - ax1) * (ay2 - ay1)

    neg_inf = jnp.float32(-jnp.inf)
    cur = jnp.full(ax1.shape, neg_inf, jnp.float32)
    mx1 = jnp.zeros(ax1.shape, jnp.float32)
    my1 = jnp.zeros(ax1.shape, jnp.float32)
    mx2 = jnp.zeros(ax1.shape, jnp.float32)
    my2 = jnp.zeros(ax1.shape, jnp.float32)

    for j in range(_NGT):
        gx1 = gt_ref[0, 0, j]
        gy1 = gt_ref[0, 1, j]
        gx2 = gt_ref[0, 2, j]
        gy2 = gt_ref[0, 3, j]
        area2 = gt_ref[0, 4, j]
        valid = gt_ref[0, 5, j]
        x1 = jnp.maximum(ax1, gx1)
        y1 = jnp.maximum(ay1, gy1)
        x2 = jnp.minimum(ax2, gx2)
        y2 = jnp.minimum(ay2, gy2)
        inter = jnp.maximum(x2 - x1, 0.0) * jnp.maximum(y2 - y1, 0.0)
        iou = inter / (area1 + area2 - inter + 1e-08)
        iou = jnp.where(valid > 0.0, iou, -1.0)
        better = iou > cur
        cur = jnp.where(better, iou, cur)
        mx1 = jnp.where(better, gx1, mx1)
        my1 = jnp.where(better, gy1, my1)
        mx2 = jnp.where(better, gx2, mx2)
        my2 = jnp.where(better, gy2, my2)

    pos = cur >= 0.7
    lab_ref[...] = pos.astype(jnp.int32)
    posf = pos.astype(jnp.float32)

    bw = ax2 - ax1 + 1.0
    bh = ay2 - ay1 + 1.0
    bcx = ax1 + 0.5 * bw
    bcy = ay1 + 0.5 * bh
    gw = mx2 - mx1 + 1.0
    gh = my2 - my1 + 1.0
    gcx = mx1 + 0.5 * gw
    gcy = my1 + 0.5 * gh
    t0 = (gcx - bcx) / bw
    t1 = (gcy - bcy) / bh
    t2 = jnp.log(gw / bw)
    t3 = jnp.log(gh / bh)

    v = jnp.stack([t0, t1, t2, t3], axis=1)
    tg_ref[...] = jnp.reshape(v, (4 * _RB, 128))[None]

    l3 = jnp.reshape(lg_ref[0], (_RB, 2, 128))
    l0 = l3[:, 0, :]
    l1 = l3[:, 1, :]
    m = jnp.maximum(l0, l1)
    lse = m + jnp.log(jnp.exp(l0 - m) + jnp.exp(l1 - m))
    ce = lse - jnp.where(pos, l1, l0)
    acc_ref[0] = acc_ref[0] + jnp.sum(ce)

    g = jnp.concatenate([t0, t1, t2, t3], axis=1)
    vl = jnp.dot(g, e4_ref[...], preferred_element_type=jnp.float32,
                 precision=lax.Precision.HIGHEST)
    d = pr_ref[0] - vl
    ad = jnp.abs(d)
    f = jnp.where(ad < 1.0, 0.5 * d * d, ad - 0.5)
    pos_x4 = jnp.dot(posf, p4_ref[...], preferred_element_type=jnp.float32)
    acc_ref[1] = acc_ref[1] + jnp.sum(f * pos_x4)
    acc_ref[2] = acc_ref[2] + jnp.sum(posf)

    cls_ref[0, 0] = acc_ref[0] / _TOTAL
    bbox_ref[0, 0] = acc_ref[1] / jnp.maximum(acc_ref[2], 1.0)


@jax.jit
def _run(rpn_cls_logits, rpn_bbox_pred, gt_boxes, gt_labels):
    anchors = _all_anchors_np()
    ax1 = jnp.asarray(anchors[:, 0].reshape(_ROWS, 128))
    ay1 = jnp.asarray(anchors[:, 1].reshape(_ROWS, 128))
    ax2 = jnp.asarray(anchors[:, 2].reshape(_ROWS, 128))
    ay2 = jnp.asarray(anchors[:, 3].reshape(_ROWS, 128))

    lg = (rpn_cls_logits.reshape(_B, _ROWS, 128, 2)
          .transpose(0, 1, 3, 2).reshape(_B, 2 * _ROWS, 128))
    pr = rpn_bbox_pred.reshape(_B, _ROWS, 512)

    area2 = (gt_boxes[:, :, 2] - gt_boxes[:, :, 0]) * (gt_boxes[:, :, 3] - gt_boxes[:, :, 1])
    valid = (gt_labels > 0).astype(jnp.float32)
    comps = jnp.stack(
        [gt_boxes[:, :, 0], gt_boxes[:, :, 1], gt_boxes[:, :, 2], gt_boxes[:, :, 3],
         area2, valid], axis=1)
    garr = jnp.pad(comps, ((0, 0), (0, 2), (0, 14)))

    e4_np, p4_np = _interleave_mats_np()
    e4 = jnp.asarray(e4_np)
    p4 = jnp.asarray(p4_np)

    anchor_spec = pl.BlockSpec((_RB, 128), lambda b, r: (r, 0))
    lab_spec = pl.BlockSpec((_RB, 128), lambda b, r: (r, b))
    lg_spec = pl.BlockSpec((1, 2 * _RB, 128), lambda b, r: (b, r, 0))
    pr_spec = pl.BlockSpec((1, _RB, 512), lambda b, r: (b, r, 0))
    tg_spec = pl.BlockSpec((1, 4 * _RB, 128), lambda b, r: (b, r, 0))
    scalar_out = pl.BlockSpec((1, 1), lambda b, r: (0, 0), memory_space=pltpu.SMEM)

    out = pl.pallas_call(
        _rpn_body,
        grid=(_B, _NB),
        in_specs=[
            pl.BlockSpec((1, 8, 64), lambda b, r: (b, 0, 0), memory_space=pltpu.SMEM),
            anchor_spec, anchor_spec, anchor_spec, anchor_spec,
            lg_spec, pr_spec,
            pl.BlockSpec((512, 512), lambda b, r: (0, 0)),
            pl.BlockSpec((128, 512), lambda b, r: (0, 0)),
        ],
        out_specs=[
            lab_spec, tg_spec,
            scalar_out, scalar_out,
        ],
        out_shape=[
            jax.ShapeDtypeStruct((_ROWS, 4 * 128), jnp.int32),
            jax.ShapeDtypeStruct((_B, 4 * _ROWS, 128), jnp.float32),
            jax.ShapeDtypeStruct((1, 1), jnp.float32),
            jax.ShapeDtypeStruct((1, 1), jnp.float32),
        ],
        scratch_shapes=[pltpu.SMEM((4,), jnp.float32)],
        compiler_params=pltpu.CompilerParams(
            dimension_semantics=("arbitrary", "arbitrary")),
    )(garr, ax1, ay1, ax2, ay2, lg, pr, e4, p4)

    lab, tg, cls_l, bbox_l = out
    rpn_labels = (lab.reshape(_ROWS, _B, 128)
                  .transpose(1, 0, 2).reshape(_B, _A))
    rpn_targets = (tg.reshape(_B, _ROWS, 4, 128)
                   .transpose(0, 1, 3, 2).reshape(_B, _A, 4))
    return cls_l[0, 0], bbox_l[0, 0], rpn_labels, rpn_targets


def kernel(rpn_cls_logits, rpn_bbox_pred, gt_boxes, gt_labels, feat_map_shape):
    return _run(rpn_cls_logits, rpn_bbox_pred, gt_boxes, gt_labels)
